# baseline (speedup 1.0000x reference)
import functools

import jax
import jax.numpy as jnp
from jax import lax
from jax.experimental import pallas as pl
from jax.experimental.pallas import tpu as pltpu
from jax.experimental.pallas import tpu_sc as plsc

NN = 10000
NE = 320000
HID = 128
OUT_DIM = 16
NLAYERS = 3

NC, NS = 2, 16
NQ = 4
QH = HID // NQ
NNP = 10240
NEP = 327680
CH = 128
NCHE = NEP // NS // CH
EPW = NEP // NS
NPT = NNP // NS
GCH = 80
CAP = 32
JUNK = NNP - 8
EAT = 24


@functools.cache
def _mesh():
    return plsc.VectorSubcoreMesh(
        core_axis_name="c", subcore_axis_name="s",
        num_cores=NC, num_subcores=NS)


def _sc_gather(table, idx):
    T, D = table.shape
    B = idx.shape[0]
    bpw = B // (NC * NS)
    nch = bpw // GCH

    @functools.partial(
        pl.kernel,
        out_type=jax.ShapeDtypeStruct((B, D), jnp.float32),
        mesh=_mesh(),
        compiler_params=pltpu.CompilerParams(use_tc_tiling_on_sc=False),
        scratch_types=[
            pltpu.VMEM((GCH,), jnp.int32),
            pltpu.VMEM((GCH, D), jnp.float32),
            pltpu.SemaphoreType.DMA,
        ],
    )
    def k(tbl, idxr, out, idx_v, rows_v, sem):
        wid = lax.axis_index("s") * NC + lax.axis_index("c")

        @pl.loop(0, nch)
        def _(j):
            base = wid * bpw + j * GCH
            pltpu.sync_copy(idxr.at[pl.ds(base, GCH)], idx_v)
            pltpu.async_copy(tbl.at[idx_v], rows_v, sem).wait()
            pltpu.sync_copy(rows_v, out.at[pl.ds(base, GCH)])

    return k(table, idx)


def _sc_edge(mode, q, ea, hbv, hc, idxp):
    write_e = mode != "last"
    out_type = []
    if write_e:
        out_type.append(jax.ShapeDtypeStruct((NC * NEP, QH), jnp.float32))
        out_type.append(jax.ShapeDtypeStruct((NC * NS, 2, QH), jnp.float32))
    out_type.append(jax.ShapeDtypeStruct((NC * NNP, 2 * QH), jnp.float32))

    ZR = 64
    l0 = mode == "l0"
    scratch = [
        [pltpu.VMEM((3, CH), jnp.int32) for _ in range(2)],
        [pltpu.VMEM((CH,), jnp.int32) for _ in range(2)],
        [pltpu.VMEM((CH, QH), jnp.float32) for _ in range(2)],
        [pltpu.VMEM((CH, 2 * QH), jnp.float32) for _ in range(2)],
        [pltpu.VMEM((CH, QH), jnp.float32) for _ in range(2)],
        [pltpu.VMEM((CAP, 2 * QH), jnp.float32) for _ in range(2)],
        [pltpu.VMEM((CAP,), jnp.int32) for _ in range(2)],
        pltpu.VMEM((CH + 16,), jnp.int32),
        [pltpu.VMEM((CH, QH), jnp.float32) for _ in range(2)],
        pltpu.VMEM((ZR, 2 * QH), jnp.float32),
        pltpu.VMEM((2, QH), jnp.float32),
        pltpu.VMEM_SHARED((NNP, 2 * QH), jnp.float32),
        [pltpu.SemaphoreType.DMA for _ in range(2)],
        [pltpu.SemaphoreType.DMA for _ in range(2)],
        [pltpu.SemaphoreType.DMA for _ in range(2)],
        [pltpu.SemaphoreType.DMA for _ in range(2)],
    ]

    def body(*refs):
        if write_e:
            (ear, hbvr, hcr, ipr,
             enr, bnpr, ndr,
             ib, dstov, eav, ghv, gcv, svv, sidx, dpad, env, zv,
             bnv, acc, sem_i, sem_g, sem_e, sem_s) = refs
        else:
            (ear, hbvr, hcr, ipr,
             ndr,
             ib, dstov, eav, ghv, gcv, svv, sidx, dpad, env, zv,
             bnv, acc, sem_i, sem_g, sem_e, sem_s) = refs
        c = lax.axis_index("c")
        s = lax.axis_index("s")
        qq = 2 * q + c

        @pl.loop(0, ZR)
        def _(i):
            for kk in range(2 * QH // 16):
                zv[i, pl.ds(kk * 16, 16)] = jnp.zeros((16,), jnp.float32)

        for t in range(NPT // ZR):
            pltpu.sync_copy(zv, acc.at[pl.ds(s * NPT + t * ZR, ZR)])
        plsc.subcore_barrier()

        def fire_idx(jj, slot):
            pltpu.async_copy(ipr.at[s * NCHE + jj], ib[slot], sem_i[slot])

        def drain_idx(slot):
            pltpu.make_async_copy(ipr.at[0], ib[slot], sem_i[slot]).wait()

        def fire_gathers(jj, slot):
            for kk in range(CH // 16):
                sl = pl.ds(kk * 16, 16)
                dstov[slot][sl] = ib[slot][1, sl] + qq * NNP
                ib[slot][0, sl] = ib[slot][0, sl] + qq * NNP
                if l0:
                    ib[slot][2, sl] = ib[slot][2, sl] + qq * EAT
            if l0:
                pltpu.async_copy(ear.at[ib[slot].at[2]], eav[slot],
                                 sem_g[slot])
            else:
                pltpu.async_copy(
                    ear.at[pl.ds(qq * NEP + s * EPW + jj * CH, CH)],
                    eav[slot], sem_g[slot])
            pltpu.async_copy(hbvr.at[ib[slot].at[0]], ghv[slot], sem_g[slot])
            pltpu.async_copy(hcr.at[dstov[slot]], gcv[slot], sem_g[slot])

        def drain_gathers(slot):
            pltpu.make_async_copy(hcr.at[pl.ds(0, CH)],
                                  eav[slot], sem_g[slot]).wait()
            pltpu.make_async_copy(hbvr.at[pl.ds(0, CH)],
                                  ghv[slot], sem_g[slot]).wait()
            pltpu.make_async_copy(hcr.at[pl.ds(0, CH)],
                                  gcv[slot], sem_g[slot]).wait()

        def drain_outs(slot):
            if write_e:
                pltpu.make_async_copy(env[slot], enr.at[pl.ds(0, CH)],
                                      sem_e[slot]).wait()
            pltpu.make_async_copy(svv[slot], acc.at[pl.ds(0, CAP)],
                                  sem_s[slot]).wait()

        zero_bn = tuple(jnp.zeros((16,), jnp.float32) for _ in range(4))
        fire_idx(0, 0)
        fire_idx(1, 1)
        drain_idx(0)
        fire_gathers(0, 0)

        def reset_sidx(b):
            for kk in range(CAP // 16):
                sidx[b][pl.ds(kk * 16, 16)] = jnp.full((16,), JUNK, jnp.int32)

        def pair(jo, tot):
            for b in range(2):
                jj2 = 2 * jo + b
                nb = 1 - b
                @pl.when(jj2 >= 2)
                def _():
                    drain_outs(b)
                drain_gathers(b)
                reset_sidx(b)
                for kk in range(CH // 16):
                    sl = pl.ds(kk * 16, 16)
                    dpad[sl] = ib[b][1, sl]
                dpad[pl.ds(CH, 16)] = jnp.zeros((16,), jnp.int32)
                lane0 = lax.iota(jnp.int32, 16) == 0

                @pl.when(jj2 + 2 < NCHE)
                def _():
                    fire_idx(jj2 + 2, b)

                @pl.when(jj2 + 1 < NCHE)
                def _():
                    drain_idx(nb)
                    fire_gathers(jj2 + 1, nb)

                def row(i, carry):
                    (k, dprev, a0, a1, a2, a3,
                     bn0, bn1, bn2, bn3) = carry
                    d = plsc.load_gather(dpad, [jnp.full((16,), i, jnp.int32)])
                    s0, s1 = pl.ds(0, 16), pl.ds(16, 16)
                    v0s, v1s = pl.ds(QH, 16), pl.ds(QH + 16, 16)
                    en0 = eav[b][i, s0] + ghv[b][i, s0] + gcv[b][i, s0]
                    en1 = eav[b][i, s1] + ghv[b][i, s1] + gcv[b][i, s1]
                    sg0 = 1.0 / (1.0 + jnp.exp(-en0))
                    sg1 = 1.0 / (1.0 + jnp.exp(-en1))
                    v0 = sg0 * ghv[b][i, v0s]
                    v1 = sg1 * ghv[b][i, v1s]
                    if write_e:
                        env[b][i, s0] = en0
                        env[b][i, s1] = en1
                        bn0 = bn0 + en0
                        bn1 = bn1 + en0 * en0
                        bn2 = bn2 + en1
                        bn3 = bn3 + en1 * en1
                    moved = jnp.max(jnp.where(d != dprev, 1, 0))
                    flush = jnp.logical_and(i > 0, moved > 0)

                    @pl.when(flush)
                    def _():
                        svv[b][k, s0] = a0
                        svv[b][k, s1] = a1
                        svv[b][k, v0s] = a2
                        svv[b][k, v1s] = a3
                        plsc.store_scatter(
                            sidx[b], [jnp.full((16,), k, jnp.int32)],
                            dprev, mask=lane0)
                    k2 = k + flush.astype(jnp.int32)

                    @pl.when(k2 == CAP)
                    def _():
                        pltpu.sync_copy(svv[b], acc.at[sidx[b]], add=True)
                        reset_sidx(b)
                    k3 = jnp.where(k2 == CAP, 0, k2)
                    a0n = jnp.where(flush, v0, a0 + v0)
                    a1n = jnp.where(flush, v1, a1 + v1)
                    a2n = jnp.where(flush, sg0, a2 + sg0)
                    a3n = jnp.where(flush, sg1, a3 + sg1)
                    return (k3, d, a0n, a1n, a2n, a3n, bn0, bn1, bn2, bn3)

                zv16 = jnp.zeros((16,), jnp.float32)
                init = (jnp.int32(0), jnp.zeros((16,), jnp.int32),
                        zv16, zv16, zv16, zv16) + tot
                (kf, df, f0, f1, f2, f3,
                 t0_, t1_, t2_, t3_) = lax.fori_loop(0, CH, row, init,
                                                     unroll=8)
                svv[b][kf, pl.ds(0, 16)] = f0
                svv[b][kf, pl.ds(16, 16)] = f1
                svv[b][kf, pl.ds(QH, 16)] = f2
                svv[b][kf, pl.ds(QH + 16, 16)] = f3
                plsc.store_scatter(
                    sidx[b], [jnp.full((16,), kf, jnp.int32)],
                    df, mask=lane0)
                if write_e:
                    pltpu.async_copy(
                        env[b],
                        enr.at[pl.ds(c * NEP + s * EPW + jj2 * CH, CH)],
                        sem_e[b])
                pltpu.async_copy(svv[b], acc.at[sidx[b]], sem_s[b], add=True)
                tot = (t0_, t1_, t2_, t3_)
            return tot

        tot = lax.fori_loop(0, NCHE // 2, pair, zero_bn)
        for b in range(2):
            drain_outs(b)

        if write_e:
            for kk in range(QH // 16):
                sl = pl.ds(kk * 16, 16)
                bnv[0, sl] = tot[2 * kk]
                bnv[1, sl] = tot[2 * kk + 1]
            pltpu.sync_copy(bnv, bnpr.at[c * NS + s])
        plsc.subcore_barrier()
        pltpu.sync_copy(acc.at[pl.ds(s * NPT, NPT)],
                        ndr.at[pl.ds(c * NNP + s * NPT, NPT)])

    k = functools.partial(
        pl.kernel, out_type=tuple(out_type), mesh=_mesh(),
        compiler_params=pltpu.CompilerParams(
            use_tc_tiling_on_sc=False, needs_layout_passes=False),
        scratch_types=scratch)(body)
    return k(ea, hbv, hc, idxp)


def _tc_node_prep(h, Bm, Cm, Vm, Um):
    NB = 8
    BR = NNP // NB

    def body(h_ref, b_ref, c_ref, v_ref, u_ref, hbv_ref, hc_ref, u_out):
        i = pl.program_id(0)
        hblk = h_ref[...]
        rows = i * BR + lax.broadcasted_iota(jnp.int32, (BR, 1), 0)
        mask = (rows < NN).astype(jnp.float32)
        hblk = hblk * mask
        prodb = jnp.dot(hblk, b_ref[...], preferred_element_type=jnp.float32)
        prodv = jnp.dot(hblk, v_ref[...], preferred_element_type=jnp.float32)
        prodc = jnp.dot(hblk, c_ref[...], preferred_element_type=jnp.float32)
        for z in range(NQ):
            hbv_ref[z, 0] = jnp.concatenate(
                [prodb[:, z * QH:(z + 1) * QH],
                 prodv[:, z * QH:(z + 1) * QH]], axis=1)
            hc_ref[z, 0] = prodc[:, z * QH:(z + 1) * QH]
        u_out[...] = jnp.dot(hblk, u_ref[...],
                             preferred_element_type=jnp.float32)

    full = lambda shape: pl.BlockSpec(shape, lambda i: (0,) * len(shape))
    outs = pl.pallas_call(
        body,
        grid=(NB,),
        in_specs=[
            pl.BlockSpec((BR, HID), lambda i: (i, 0)),
            full((HID, HID)), full((HID, HID)), full((HID, HID)),
            full((HID, HID)),
        ],
        out_specs=[
            pl.BlockSpec((NQ, 1, BR, 2 * QH), lambda i: (0, i, 0, 0)),
            pl.BlockSpec((NQ, 1, BR, QH), lambda i: (0, i, 0, 0)),
            pl.BlockSpec((BR, HID), lambda i: (i, 0)),
        ],
        out_shape=[
            jax.ShapeDtypeStruct((NQ, NB, BR, 2 * QH), jnp.float32),
            jax.ShapeDtypeStruct((NQ, NB, BR, QH), jnp.float32),
            jax.ShapeDtypeStruct((NNP, HID), jnp.float32),
        ],
    )(h, Bm, Cm, Vm, Um)
    hbv, hc, hu = outs
    return hbv.reshape(NQ * NNP, 2 * QH), hc.reshape(NQ * NNP, QH), hu


def _tc_ea0_dense(e_emb, A0, efp):
    EB = 512
    NBLK = NEP // EB

    def body(ef_ref, e_ref, a_ref, out_ref):
        prod = jnp.dot(e_ref[...], a_ref[...],
                       preferred_element_type=jnp.float32)
        tbl = jnp.concatenate(
            [prod, jnp.zeros((1, HID), jnp.float32)], axis=0)
        ids = ef_ref[0, 0]
        oh = (ids[:, None] == lax.broadcasted_iota(jnp.int32, (1, 17), 1)
              ).astype(jnp.float32)
        eo = jnp.dot(oh, tbl, preferred_element_type=jnp.float32)
        for z in range(NQ):
            out_ref[z, 0] = eo[:, z * QH:(z + 1) * QH]

    full = lambda shape: pl.BlockSpec(shape, lambda i: (0,) * len(shape))
    out = pl.pallas_call(
        body,
        grid=(NBLK,),
        in_specs=[
            pl.BlockSpec((1, 1, EB), lambda i: (i, 0, 0)),
            full((16, HID)), full((HID, HID)),
        ],
        out_specs=pl.BlockSpec((NQ, 1, EB, QH), lambda i: (0, i, 0, 0)),
        out_shape=jax.ShapeDtypeStruct((NQ, NBLK, EB, QH), jnp.float32),
    )(efp.reshape(NBLK, 1, EB), e_emb, A0)
    return out.reshape(NQ * NEP, QH)


def _tc_node_update(hu, nd0, nd1, h_cur, gh_l, bh_l):
    def body(hu_ref, nd0_ref, nd1_ref, h_ref, g_ref, b_ref, out_ref):
        nd0v, nd1v = nd0_ref[...], nd1_ref[...]
        num = jnp.concatenate(
            [nd0v[0, :, :QH], nd0v[1, :, :QH],
             nd1v[0, :, :QH], nd1v[1, :, :QH]], axis=1)
        den = jnp.concatenate(
            [nd0v[0, :, QH:], nd0v[1, :, QH:],
             nd1v[0, :, QH:], nd1v[1, :, QH:]], axis=1)
        hn = hu_ref[...] + num / (den + 1e-6)
        rows = lax.broadcasted_iota(jnp.int32, (NNP, 1), 0)
        maskf = (rows < NN).astype(jnp.float32)
        hn = hn * maskf
        mu = jnp.sum(hn, axis=0, keepdims=True) / NN
        msq = jnp.sum(hn * hn, axis=0, keepdims=True) / NN
        var = msq - mu * mu
        hb = (hn - mu) * lax.rsqrt(var + 1e-5) * g_ref[...] + b_ref[...]
        out_ref[...] = (h_ref[...] + jnp.maximum(hb, 0.0)) * maskf

    return pl.pallas_call(
        body,
        out_shape=jax.ShapeDtypeStruct((NNP, HID), jnp.float32),
    )(hu, nd0.reshape(NC, NNP, 2 * QH), nd1.reshape(NC, NNP, 2 * QH), h_cur,
      gh_l.reshape(1, HID), bh_l.reshape(1, HID))


def _tc_edge_finish(l0, ecur_or_ef, e_emb, en0, en1, bnp0, bnp1,
                    ge_l, be_l, A_next):
    EB = 512
    NBLK = NEP // EB
    REAL = NE // EB

    def body(ec_ref, emb_ref, en0_ref, en1_ref, b0_ref, b1_ref,
             g_ref, b_ref, a_ref, eo_ref, ea_ref):
        i = pl.program_id(0)
        b0 = b0_ref[...]
        b1 = b1_ref[...]
        sums = jnp.concatenate(
            [jnp.sum(b0[:NS, 0, :], axis=0), jnp.sum(b0[NS:, 0, :], axis=0),
             jnp.sum(b1[:NS, 0, :], axis=0), jnp.sum(b1[NS:, 0, :], axis=0)])
        sqs = jnp.concatenate(
            [jnp.sum(b0[:NS, 1, :], axis=0), jnp.sum(b0[NS:, 1, :], axis=0),
             jnp.sum(b1[:NS, 1, :], axis=0), jnp.sum(b1[NS:, 1, :], axis=0)])
        mu = (sums / NE).reshape(1, HID)
        var = (sqs / NE).reshape(1, HID) - mu * mu
        scale = lax.rsqrt(var + 1e-5) * g_ref[...]
        shift = b_ref[...] - mu * scale

        @pl.when(i < REAL)
        def _():
            en = jnp.concatenate(
                [en0_ref[0, 0], en0_ref[1, 0], en1_ref[0, 0], en1_ref[1, 0]],
                axis=1)
            if l0:
                ids = ec_ref[0, 0]
                oh = (ids[:, None] ==
                      lax.broadcasted_iota(jnp.int32, (1, 16), 1)
                      ).astype(jnp.float32)
                ecur = jnp.dot(oh, emb_ref[...],
                               preferred_element_type=jnp.float32)
            else:
                ecur = ec_ref[...]
            eo = ecur + jnp.maximum(en * scale + shift, 0.0)
            eo_ref[...] = eo
            ea = jnp.dot(eo, a_ref[...], preferred_element_type=jnp.float32)
            for z in range(NQ):
                ea_ref[z, 0] = ea[:, z * QH:(z + 1) * QH]

        @pl.when(i >= REAL)
        def _():
            eo_ref[...] = jnp.zeros((EB, HID), jnp.float32)
            ea_ref[...] = jnp.zeros((NQ, 1, EB, QH), jnp.float32)

    if l0:
        ec_spec = pl.BlockSpec((1, 1, EB), lambda i: (i, 0, 0))
        ec_arg = ecur_or_ef.reshape(NBLK, 1, EB)
    else:
        ec_spec = pl.BlockSpec((EB, HID), lambda i: (i, 0))
        ec_arg = ecur_or_ef
    full = lambda shape: pl.BlockSpec(shape, lambda i: (0,) * len(shape))
    eo, ea = pl.pallas_call(
        body,
        grid=(NBLK,),
        in_specs=[
            ec_spec,
            full((16, HID)),
            pl.BlockSpec((NC, 1, EB, QH), lambda i: (0, i, 0, 0)),
            pl.BlockSpec((NC, 1, EB, QH), lambda i: (0, i, 0, 0)),
            full((NC * NS, 2, QH)), full((NC * NS, 2, QH)),
            full((1, HID)), full((1, HID)), full((HID, HID)),
        ],
        out_specs=[
            pl.BlockSpec((EB, HID), lambda i: (i, 0)),
            pl.BlockSpec((NQ, 1, EB, QH), lambda i: (0, i, 0, 0)),
        ],
        out_shape=[
            jax.ShapeDtypeStruct((NEP, HID), jnp.float32),
            jax.ShapeDtypeStruct((NQ, NBLK, EB, QH), jnp.float32),
        ],
    )(ec_arg, e_emb, en0.reshape(NC, NBLK, EB, QH),
      en1.reshape(NC, NBLK, EB, QH), bnp0, bnp1,
      ge_l.reshape(1, HID), be_l.reshape(1, HID), A_next)
    return eo, ea.reshape(NQ * NEP, QH)


def _tc_mlp(ht0, ht2, W1a, W1b, b1, W2, b2):
    def body(h0_ref, h2_ref, wa_ref, wb_ref, b1_ref, w2_ref, b2_ref, out_ref):
        z = (jnp.dot(h0_ref[...], wa_ref[...],
                     preferred_element_type=jnp.float32)
             + jnp.dot(h2_ref[...], wb_ref[...],
                       preferred_element_type=jnp.float32)
             + b1_ref[...])
        z = jnp.maximum(z, 0.0)
        out_ref[...] = jnp.dot(z, w2_ref[...],
                               preferred_element_type=jnp.float32) + b2_ref[...]

    return pl.pallas_call(
        body,
        out_shape=jax.ShapeDtypeStruct((NN, OUT_DIM), jnp.float32),
    )(ht0, ht2, W1a, W1b, b1.reshape(1, HID), W2, b2.reshape(1, OUT_DIM))


def kernel(node_feat, edge_index, edge_feat, triplets, h_emb, e_emb,
           A, B, C, U, V, gh, bh, ge, be, W1, b1, W2, b2):
    i32 = jnp.int32
    PADN = NN + 16

    dst_u = edge_index[1].astype(i32)
    perm = jnp.argsort(dst_u)
    src = edge_index[0].astype(i32)[perm]
    dst = dst_u[perm]
    efs = edge_feat.astype(i32)[perm]
    srcp = jnp.full((NEP,), PADN, i32).at[:NE].set(src)
    dstp = jnp.full((NEP,), PADN, i32).at[:NE].set(dst)
    efp = jnp.full((NEP,), 16, i32).at[:NE].set(efs)
    idxp = jnp.stack([srcp.reshape(NEP // CH, CH), dstp.reshape(NEP // CH, CH),
                      efp.reshape(NEP // CH, CH)], axis=1)
    nfp = jnp.zeros((NNP,), i32).at[:NN].set(node_feat.astype(i32))
    t0 = jnp.zeros((NNP,), i32).at[:NN].set(triplets[:, 0].astype(i32))
    t2 = jnp.zeros((NNP,), i32).at[:NN].set(triplets[:, 2].astype(i32))
    tidx = jnp.concatenate([t0, t2])

    h = _sc_gather(h_emb, nfp)
    ea = _tc_ea0_dense(e_emb, A[0], efp)

    ecur = None
    for l in range(NLAYERS):
        hbv, hc, hu = _tc_node_prep(h, B[l], C[l], V[l], U[l])
        mode = "last" if l == NLAYERS - 1 else "mid"
        r0 = _sc_edge(mode, 0, ea, hbv, hc, idxp)
        r1 = _sc_edge(mode, 1, ea, hbv, hc, idxp)
        if mode == "last":
            nd0, nd1 = r0[0], r1[0]
            en0 = en1 = bnp0 = bnp1 = None
        else:
            en0, bnp0, nd0 = r0
            en1, bnp1, nd1 = r1
        h = _tc_node_update(hu, nd0, nd1, h, gh[l], bh[l])
        if l < NLAYERS - 1:
            ecur, ea = _tc_edge_finish(
                l == 0, efp if l == 0 else ecur, e_emb, en0, en1, bnp0, bnp1,
                ge[l], be[l], A[l + 1])

    ht = _sc_gather(h, tidx)
    score = _tc_mlp(ht[:NN], ht[NNP:NNP + NN],
                    W1[:HID], W1[HID:], b1, W2, b2)
    return score


# speedup vs baseline: 1.0256x; 1.0256x over previous
import functools

import jax
import jax.numpy as jnp
from jax import lax
from jax.experimental import pallas as pl
from jax.experimental.pallas import tpu as pltpu
from jax.experimental.pallas import tpu_sc as plsc

NN = 10000
NE = 320000
HID = 128
OUT_DIM = 16
NLAYERS = 3

NC, NS = 2, 16
NQ = 4
QH = HID // NQ
NNP = 10240
NEP = 327680
CH = 128
NCHE = NEP // NS // CH
EPW = NEP // NS
NPT = NNP // NS
GCH = 80
CAP = 32
JUNK = NNP - 8
EAT = 24


@functools.cache
def _mesh():
    return plsc.VectorSubcoreMesh(
        core_axis_name="c", subcore_axis_name="s",
        num_cores=NC, num_subcores=NS)


def _sc_gather(table, idx):
    T, D = table.shape
    B = idx.shape[0]
    bpw = B // (NC * NS)
    nch = bpw // GCH

    @functools.partial(
        pl.kernel,
        out_type=jax.ShapeDtypeStruct((B, D), jnp.float32),
        mesh=_mesh(),
        compiler_params=pltpu.CompilerParams(use_tc_tiling_on_sc=False),
        scratch_types=[
            pltpu.VMEM((GCH,), jnp.int32),
            pltpu.VMEM((GCH, D), jnp.float32),
            pltpu.SemaphoreType.DMA,
        ],
    )
    def k(tbl, idxr, out, idx_v, rows_v, sem):
        wid = lax.axis_index("s") * NC + lax.axis_index("c")

        @pl.loop(0, nch)
        def _(j):
            base = wid * bpw + j * GCH
            pltpu.sync_copy(idxr.at[pl.ds(base, GCH)], idx_v)
            pltpu.async_copy(tbl.at[idx_v], rows_v, sem).wait()
            pltpu.sync_copy(rows_v, out.at[pl.ds(base, GCH)])

    return k(table, idx)


def _sc_edge(mode, q, ea, hbv, hc, idxp):
    write_e = mode != "last"
    out_type = []
    if write_e:
        out_type.append(jax.ShapeDtypeStruct((NC * NEP, QH), jnp.float32))
        out_type.append(jax.ShapeDtypeStruct((NC * NS, 2, QH), jnp.float32))
    out_type.append(jax.ShapeDtypeStruct((NC * NNP, 2 * QH), jnp.float32))

    ZR = 64
    l0 = mode == "l0"
    scratch = [
        [pltpu.VMEM((3, CH), jnp.int32) for _ in range(2)],
        [pltpu.VMEM((CH,), jnp.int32) for _ in range(2)],
        [pltpu.VMEM((CH, QH), jnp.float32) for _ in range(2)],
        [pltpu.VMEM((CH, 2 * QH), jnp.float32) for _ in range(2)],
        [pltpu.VMEM((CH, QH), jnp.float32) for _ in range(2)],
        [pltpu.VMEM((CAP, 2 * QH), jnp.float32) for _ in range(2)],
        [pltpu.VMEM((CAP,), jnp.int32) for _ in range(2)],
        pltpu.VMEM((CH + 16,), jnp.int32),
        [pltpu.VMEM((CH, QH), jnp.float32) for _ in range(2)],
        pltpu.VMEM((ZR, 2 * QH), jnp.float32),
        pltpu.VMEM((2, QH), jnp.float32),
        pltpu.VMEM_SHARED((NNP, 2 * QH), jnp.float32),
        [pltpu.SemaphoreType.DMA for _ in range(2)],
        [pltpu.SemaphoreType.DMA for _ in range(2)],
        [pltpu.SemaphoreType.DMA for _ in range(2)],
        [pltpu.SemaphoreType.DMA for _ in range(2)],
    ]

    def body(*refs):
        if write_e:
            (ear, hbvr, hcr, ipr,
             enr, bnpr, ndr,
             ib, dstov, eav, ghv, gcv, svv, sidx, dpad, env, zv,
             bnv, acc, sem_i, sem_g, sem_e, sem_s) = refs
        else:
            (ear, hbvr, hcr, ipr,
             ndr,
             ib, dstov, eav, ghv, gcv, svv, sidx, dpad, env, zv,
             bnv, acc, sem_i, sem_g, sem_e, sem_s) = refs
        c = lax.axis_index("c")
        s = lax.axis_index("s")
        qq = 2 * q + c

        @pl.loop(0, ZR)
        def _(i):
            for kk in range(2 * QH // 16):
                zv[i, pl.ds(kk * 16, 16)] = jnp.zeros((16,), jnp.float32)

        for t in range(NPT // ZR):
            pltpu.sync_copy(zv, acc.at[pl.ds(s * NPT + t * ZR, ZR)])
        plsc.subcore_barrier()

        def fire_idx(jj, slot):
            pltpu.async_copy(ipr.at[s * NCHE + jj], ib[slot], sem_i[slot])

        def drain_idx(slot):
            pltpu.make_async_copy(ipr.at[0], ib[slot], sem_i[slot]).wait()

        def fire_gathers(jj, slot):
            for kk in range(CH // 16):
                sl = pl.ds(kk * 16, 16)
                dstov[slot][sl] = ib[slot][1, sl] + qq * NNP
                ib[slot][0, sl] = ib[slot][0, sl] + qq * NNP
                if l0:
                    ib[slot][2, sl] = ib[slot][2, sl] + qq * EAT
            if l0:
                pltpu.async_copy(ear.at[ib[slot].at[2]], eav[slot],
                                 sem_g[slot])
            else:
                pltpu.async_copy(
                    ear.at[pl.ds(qq * NEP + s * EPW + jj * CH, CH)],
                    eav[slot], sem_g[slot])
            pltpu.async_copy(hbvr.at[ib[slot].at[0]], ghv[slot], sem_g[slot])
            pltpu.async_copy(hcr.at[dstov[slot]], gcv[slot], sem_g[slot])

        def drain_gathers(slot):
            pltpu.make_async_copy(hcr.at[pl.ds(0, CH)],
                                  eav[slot], sem_g[slot]).wait()
            pltpu.make_async_copy(hbvr.at[pl.ds(0, CH)],
                                  ghv[slot], sem_g[slot]).wait()
            pltpu.make_async_copy(hcr.at[pl.ds(0, CH)],
                                  gcv[slot], sem_g[slot]).wait()

        def drain_outs(slot):
            if write_e:
                pltpu.make_async_copy(env[slot], enr.at[pl.ds(0, CH)],
                                      sem_e[slot]).wait()
            pltpu.make_async_copy(svv[slot], acc.at[pl.ds(0, CAP)],
                                  sem_s[slot]).wait()

        zero_bn = tuple(jnp.zeros((16,), jnp.float32) for _ in range(4))
        fire_idx(0, 0)
        fire_idx(1, 1)
        drain_idx(0)
        fire_gathers(0, 0)

        def reset_sidx(b):
            for kk in range(CAP // 16):
                sidx[b][pl.ds(kk * 16, 16)] = jnp.full((16,), JUNK, jnp.int32)

        def pair(jo, tot):
            for b in range(2):
                jj2 = 2 * jo + b
                nb = 1 - b
                @pl.when(jj2 >= 2)
                def _():
                    drain_outs(b)
                drain_gathers(b)
                reset_sidx(b)
                for kk in range(CH // 16):
                    sl = pl.ds(kk * 16, 16)
                    dpad[sl] = ib[b][1, sl]
                dpad[pl.ds(CH, 16)] = jnp.zeros((16,), jnp.int32)
                lane0 = lax.iota(jnp.int32, 16) == 0

                @pl.when(jj2 + 2 < NCHE)
                def _():
                    fire_idx(jj2 + 2, b)

                @pl.when(jj2 + 1 < NCHE)
                def _():
                    drain_idx(nb)
                    fire_gathers(jj2 + 1, nb)

                def row(i, carry):
                    (k, dprev, a0, a1, a2, a3,
                     bn0, bn1, bn2, bn3) = carry
                    d = plsc.load_gather(dpad, [jnp.full((16,), i, jnp.int32)])
                    s0, s1 = pl.ds(0, 16), pl.ds(16, 16)
                    v0s, v1s = pl.ds(QH, 16), pl.ds(QH + 16, 16)
                    en0 = eav[b][i, s0] + ghv[b][i, s0] + gcv[b][i, s0]
                    en1 = eav[b][i, s1] + ghv[b][i, s1] + gcv[b][i, s1]
                    sg0 = 1.0 / (1.0 + jnp.exp(-en0))
                    sg1 = 1.0 / (1.0 + jnp.exp(-en1))
                    v0 = sg0 * ghv[b][i, v0s]
                    v1 = sg1 * ghv[b][i, v1s]
                    if write_e:
                        env[b][i, s0] = en0
                        env[b][i, s1] = en1
                        bn0 = bn0 + en0
                        bn1 = bn1 + en0 * en0
                        bn2 = bn2 + en1
                        bn3 = bn3 + en1 * en1
                    moved = jnp.max(jnp.where(d != dprev, 1, 0))
                    flush = jnp.logical_and(i > 0, moved > 0)

                    @pl.when(flush)
                    def _():
                        svv[b][k, s0] = a0
                        svv[b][k, s1] = a1
                        svv[b][k, v0s] = a2
                        svv[b][k, v1s] = a3
                        plsc.store_scatter(
                            sidx[b], [jnp.full((16,), k, jnp.int32)],
                            dprev, mask=lane0)
                    k2 = k + flush.astype(jnp.int32)

                    @pl.when(k2 == CAP)
                    def _():
                        pltpu.sync_copy(svv[b], acc.at[sidx[b]], add=True)
                        reset_sidx(b)
                    k3 = jnp.where(k2 == CAP, 0, k2)
                    a0n = jnp.where(flush, v0, a0 + v0)
                    a1n = jnp.where(flush, v1, a1 + v1)
                    a2n = jnp.where(flush, sg0, a2 + sg0)
                    a3n = jnp.where(flush, sg1, a3 + sg1)
                    return (k3, d, a0n, a1n, a2n, a3n, bn0, bn1, bn2, bn3)

                zv16 = jnp.zeros((16,), jnp.float32)
                init = (jnp.int32(0), jnp.zeros((16,), jnp.int32),
                        zv16, zv16, zv16, zv16) + tot
                (kf, df, f0, f1, f2, f3,
                 t0_, t1_, t2_, t3_) = lax.fori_loop(0, CH, row, init,
                                                     unroll=4)
                svv[b][kf, pl.ds(0, 16)] = f0
                svv[b][kf, pl.ds(16, 16)] = f1
                svv[b][kf, pl.ds(QH, 16)] = f2
                svv[b][kf, pl.ds(QH + 16, 16)] = f3
                plsc.store_scatter(
                    sidx[b], [jnp.full((16,), kf, jnp.int32)],
                    df, mask=lane0)
                if write_e:
                    pltpu.async_copy(
                        env[b],
                        enr.at[pl.ds(c * NEP + s * EPW + jj2 * CH, CH)],
                        sem_e[b])
                pltpu.async_copy(svv[b], acc.at[sidx[b]], sem_s[b], add=True)
                tot = (t0_, t1_, t2_, t3_)
            return tot

        tot = lax.fori_loop(0, NCHE // 2, pair, zero_bn)
        for b in range(2):
            drain_outs(b)

        if write_e:
            for kk in range(QH // 16):
                sl = pl.ds(kk * 16, 16)
                bnv[0, sl] = tot[2 * kk]
                bnv[1, sl] = tot[2 * kk + 1]
            pltpu.sync_copy(bnv, bnpr.at[c * NS + s])
        plsc.subcore_barrier()
        pltpu.sync_copy(acc.at[pl.ds(s * NPT, NPT)],
                        ndr.at[pl.ds(c * NNP + s * NPT, NPT)])

    k = functools.partial(
        pl.kernel, out_type=tuple(out_type), mesh=_mesh(),
        compiler_params=pltpu.CompilerParams(
            use_tc_tiling_on_sc=False, needs_layout_passes=False),
        scratch_types=scratch)(body)
    return k(ea, hbv, hc, idxp)


def _tc_node_prep(h, Bm, Cm, Vm, Um):
    NB = 8
    BR = NNP // NB

    def body(h_ref, b_ref, c_ref, v_ref, u_ref, hbv_ref, hc_ref, u_out):
        i = pl.program_id(0)
        hblk = h_ref[...]
        rows = i * BR + lax.broadcasted_iota(jnp.int32, (BR, 1), 0)
        mask = (rows < NN).astype(jnp.float32)
        hblk = hblk * mask
        prodb = jnp.dot(hblk, b_ref[...], preferred_element_type=jnp.float32)
        prodv = jnp.dot(hblk, v_ref[...], preferred_element_type=jnp.float32)
        prodc = jnp.dot(hblk, c_ref[...], preferred_element_type=jnp.float32)
        for z in range(NQ):
            hbv_ref[z, 0] = jnp.concatenate(
                [prodb[:, z * QH:(z + 1) * QH],
                 prodv[:, z * QH:(z + 1) * QH]], axis=1)
            hc_ref[z, 0] = prodc[:, z * QH:(z + 1) * QH]
        u_out[...] = jnp.dot(hblk, u_ref[...],
                             preferred_element_type=jnp.float32)

    full = lambda shape: pl.BlockSpec(shape, lambda i: (0,) * len(shape))
    outs = pl.pallas_call(
        body,
        grid=(NB,),
        in_specs=[
            pl.BlockSpec((BR, HID), lambda i: (i, 0)),
            full((HID, HID)), full((HID, HID)), full((HID, HID)),
            full((HID, HID)),
        ],
        out_specs=[
            pl.BlockSpec((NQ, 1, BR, 2 * QH), lambda i: (0, i, 0, 0)),
            pl.BlockSpec((NQ, 1, BR, QH), lambda i: (0, i, 0, 0)),
            pl.BlockSpec((BR, HID), lambda i: (i, 0)),
        ],
        out_shape=[
            jax.ShapeDtypeStruct((NQ, NB, BR, 2 * QH), jnp.float32),
            jax.ShapeDtypeStruct((NQ, NB, BR, QH), jnp.float32),
            jax.ShapeDtypeStruct((NNP, HID), jnp.float32),
        ],
    )(h, Bm, Cm, Vm, Um)
    hbv, hc, hu = outs
    return hbv.reshape(NQ * NNP, 2 * QH), hc.reshape(NQ * NNP, QH), hu


def _tc_ea0_dense(e_emb, A0, efp):
    EB = 512
    NBLK = NEP // EB

    def body(ef_ref, e_ref, a_ref, out_ref):
        prod = jnp.dot(e_ref[...], a_ref[...],
                       preferred_element_type=jnp.float32)
        tbl = jnp.concatenate(
            [prod, jnp.zeros((1, HID), jnp.float32)], axis=0)
        ids = ef_ref[0, 0]
        oh = (ids[:, None] == lax.broadcasted_iota(jnp.int32, (1, 17), 1)
              ).astype(jnp.float32)
        eo = jnp.dot(oh, tbl, preferred_element_type=jnp.float32)
        for z in range(NQ):
            out_ref[z, 0] = eo[:, z * QH:(z + 1) * QH]

    full = lambda shape: pl.BlockSpec(shape, lambda i: (0,) * len(shape))
    out = pl.pallas_call(
        body,
        grid=(NBLK,),
        in_specs=[
            pl.BlockSpec((1, 1, EB), lambda i: (i, 0, 0)),
            full((16, HID)), full((HID, HID)),
        ],
        out_specs=pl.BlockSpec((NQ, 1, EB, QH), lambda i: (0, i, 0, 0)),
        out_shape=jax.ShapeDtypeStruct((NQ, NBLK, EB, QH), jnp.float32),
    )(efp.reshape(NBLK, 1, EB), e_emb, A0)
    return out.reshape(NQ * NEP, QH)


def _tc_node_update(hu, nd0, nd1, h_cur, gh_l, bh_l):
    def body(hu_ref, nd0_ref, nd1_ref, h_ref, g_ref, b_ref, out_ref):
        nd0v, nd1v = nd0_ref[...], nd1_ref[...]
        num = jnp.concatenate(
            [nd0v[0, :, :QH], nd0v[1, :, :QH],
             nd1v[0, :, :QH], nd1v[1, :, :QH]], axis=1)
        den = jnp.concatenate(
            [nd0v[0, :, QH:], nd0v[1, :, QH:],
             nd1v[0, :, QH:], nd1v[1, :, QH:]], axis=1)
        hn = hu_ref[...] + num / (den + 1e-6)
        rows = lax.broadcasted_iota(jnp.int32, (NNP, 1), 0)
        maskf = (rows < NN).astype(jnp.float32)
        hn = hn * maskf
        mu = jnp.sum(hn, axis=0, keepdims=True) / NN
        msq = jnp.sum(hn * hn, axis=0, keepdims=True) / NN
        var = msq - mu * mu
        hb = (hn - mu) * lax.rsqrt(var + 1e-5) * g_ref[...] + b_ref[...]
        out_ref[...] = (h_ref[...] + jnp.maximum(hb, 0.0)) * maskf

    return pl.pallas_call(
        body,
        out_shape=jax.ShapeDtypeStruct((NNP, HID), jnp.float32),
    )(hu, nd0.reshape(NC, NNP, 2 * QH), nd1.reshape(NC, NNP, 2 * QH), h_cur,
      gh_l.reshape(1, HID), bh_l.reshape(1, HID))


def _tc_edge_finish(l0, ecur_or_ef, e_emb, en0, en1, bnp0, bnp1,
                    ge_l, be_l, A_next):
    EB = 512
    NBLK = NEP // EB
    REAL = NE // EB

    def body(ec_ref, emb_ref, en0_ref, en1_ref, b0_ref, b1_ref,
             g_ref, b_ref, a_ref, eo_ref, ea_ref):
        i = pl.program_id(0)
        b0 = b0_ref[...]
        b1 = b1_ref[...]
        sums = jnp.concatenate(
            [jnp.sum(b0[:NS, 0, :], axis=0), jnp.sum(b0[NS:, 0, :], axis=0),
             jnp.sum(b1[:NS, 0, :], axis=0), jnp.sum(b1[NS:, 0, :], axis=0)])
        sqs = jnp.concatenate(
            [jnp.sum(b0[:NS, 1, :], axis=0), jnp.sum(b0[NS:, 1, :], axis=0),
             jnp.sum(b1[:NS, 1, :], axis=0), jnp.sum(b1[NS:, 1, :], axis=0)])
        mu = (sums / NE).reshape(1, HID)
        var = (sqs / NE).reshape(1, HID) - mu * mu
        scale = lax.rsqrt(var + 1e-5) * g_ref[...]
        shift = b_ref[...] - mu * scale

        @pl.when(i < REAL)
        def _():
            en = jnp.concatenate(
                [en0_ref[0, 0], en0_ref[1, 0], en1_ref[0, 0], en1_ref[1, 0]],
                axis=1)
            if l0:
                ids = ec_ref[0, 0]
                oh = (ids[:, None] ==
                      lax.broadcasted_iota(jnp.int32, (1, 16), 1)
                      ).astype(jnp.float32)
                ecur = jnp.dot(oh, emb_ref[...],
                               preferred_element_type=jnp.float32)
            else:
                ecur = ec_ref[...]
            eo = ecur + jnp.maximum(en * scale + shift, 0.0)
            eo_ref[...] = eo
            ea = jnp.dot(eo, a_ref[...], preferred_element_type=jnp.float32)
            for z in range(NQ):
                ea_ref[z, 0] = ea[:, z * QH:(z + 1) * QH]

        @pl.when(i >= REAL)
        def _():
            eo_ref[...] = jnp.zeros((EB, HID), jnp.float32)
            ea_ref[...] = jnp.zeros((NQ, 1, EB, QH), jnp.float32)

    if l0:
        ec_spec = pl.BlockSpec((1, 1, EB), lambda i: (i, 0, 0))
        ec_arg = ecur_or_ef.reshape(NBLK, 1, EB)
    else:
        ec_spec = pl.BlockSpec((EB, HID), lambda i: (i, 0))
        ec_arg = ecur_or_ef
    full = lambda shape: pl.BlockSpec(shape, lambda i: (0,) * len(shape))
    eo, ea = pl.pallas_call(
        body,
        grid=(NBLK,),
        in_specs=[
            ec_spec,
            full((16, HID)),
            pl.BlockSpec((NC, 1, EB, QH), lambda i: (0, i, 0, 0)),
            pl.BlockSpec((NC, 1, EB, QH), lambda i: (0, i, 0, 0)),
            full((NC * NS, 2, QH)), full((NC * NS, 2, QH)),
            full((1, HID)), full((1, HID)), full((HID, HID)),
        ],
        out_specs=[
            pl.BlockSpec((EB, HID), lambda i: (i, 0)),
            pl.BlockSpec((NQ, 1, EB, QH), lambda i: (0, i, 0, 0)),
        ],
        out_shape=[
            jax.ShapeDtypeStruct((NEP, HID), jnp.float32),
            jax.ShapeDtypeStruct((NQ, NBLK, EB, QH), jnp.float32),
        ],
    )(ec_arg, e_emb, en0.reshape(NC, NBLK, EB, QH),
      en1.reshape(NC, NBLK, EB, QH), bnp0, bnp1,
      ge_l.reshape(1, HID), be_l.reshape(1, HID), A_next)
    return eo, ea.reshape(NQ * NEP, QH)


def _tc_mlp(ht0, ht2, W1a, W1b, b1, W2, b2):
    def body(h0_ref, h2_ref, wa_ref, wb_ref, b1_ref, w2_ref, b2_ref, out_ref):
        z = (jnp.dot(h0_ref[...], wa_ref[...],
                     preferred_element_type=jnp.float32)
             + jnp.dot(h2_ref[...], wb_ref[...],
                       preferred_element_type=jnp.float32)
             + b1_ref[...])
        z = jnp.maximum(z, 0.0)
        out_ref[...] = jnp.dot(z, w2_ref[...],
                               preferred_element_type=jnp.float32) + b2_ref[...]

    return pl.pallas_call(
        body,
        out_shape=jax.ShapeDtypeStruct((NN, OUT_DIM), jnp.float32),
    )(ht0, ht2, W1a, W1b, b1.reshape(1, HID), W2, b2.reshape(1, OUT_DIM))


def kernel(node_feat, edge_index, edge_feat, triplets, h_emb, e_emb,
           A, B, C, U, V, gh, bh, ge, be, W1, b1, W2, b2):
    i32 = jnp.int32
    PADN = NN + 16

    dst_u = edge_index[1].astype(i32)
    perm = jnp.argsort(dst_u)
    src = edge_index[0].astype(i32)[perm]
    dst = dst_u[perm]
    efs = edge_feat.astype(i32)[perm]
    srcp = jnp.full((NEP,), PADN, i32).at[:NE].set(src)
    dstp = jnp.full((NEP,), PADN, i32).at[:NE].set(dst)
    efp = jnp.full((NEP,), 16, i32).at[:NE].set(efs)
    idxp = jnp.stack([srcp.reshape(NEP // CH, CH), dstp.reshape(NEP // CH, CH),
                      efp.reshape(NEP // CH, CH)], axis=1)
    nfp = jnp.zeros((NNP,), i32).at[:NN].set(node_feat.astype(i32))
    t0 = jnp.zeros((NNP,), i32).at[:NN].set(triplets[:, 0].astype(i32))
    t2 = jnp.zeros((NNP,), i32).at[:NN].set(triplets[:, 2].astype(i32))
    tidx = jnp.concatenate([t0, t2])

    h = _sc_gather(h_emb, nfp)
    ea = _tc_ea0_dense(e_emb, A[0], efp)

    ecur = None
    for l in range(NLAYERS):
        hbv, hc, hu = _tc_node_prep(h, B[l], C[l], V[l], U[l])
        mode = "last" if l == NLAYERS - 1 else "mid"
        r0 = _sc_edge(mode, 0, ea, hbv, hc, idxp)
        r1 = _sc_edge(mode, 1, ea, hbv, hc, idxp)
        if mode == "last":
            nd0, nd1 = r0[0], r1[0]
            en0 = en1 = bnp0 = bnp1 = None
        else:
            en0, bnp0, nd0 = r0
            en1, bnp1, nd1 = r1
        h = _tc_node_update(hu, nd0, nd1, h, gh[l], bh[l])
        if l < NLAYERS - 1:
            ecur, ea = _tc_edge_finish(
                l == 0, efp if l == 0 else ecur, e_emb, en0, en1, bnp0, bnp1,
                ge[l], be[l], A[l + 1])

    ht = _sc_gather(h, tidx)
    score = _tc_mlp(ht[:NN], ht[NNP:NNP + NN],
                    W1[:HID], W1[HID:], b1, W2, b2)
    return score
